# baseline (speedup 1.0000x reference)
import jax
import jax.numpy as jnp
from jax.experimental import pallas as pl
from jax.experimental.pallas import tpu as pltpu

_BLOCK_COLS = 59392


def _copy_body(src_ref, dst_ref):
    dst_ref[...] = src_ref[...]


def kernel(inputs, ent_embeds):
    del inputs
    rows, dim = ent_embeds.shape
    wide = ent_embeds.T
    grid = (rows + _BLOCK_COLS - 1) // _BLOCK_COLS
    out = pl.pallas_call(
        _copy_body,
        out_shape=jax.ShapeDtypeStruct(wide.shape, wide.dtype),
        grid=(grid,),
        in_specs=[pl.BlockSpec((dim, _BLOCK_COLS), lambda i: (0, i))],
        out_specs=pl.BlockSpec((dim, _BLOCK_COLS), lambda i: (0, i)),
    )(wide)
    return out.T


# speedup vs baseline: 1.0017x; 1.0017x over previous
import jax
import jax.numpy as jnp
from jax.experimental import pallas as pl
from jax.experimental.pallas import tpu as pltpu

_BLOCK_COLS = 57344


def _copy_body(src_ref, dst_ref):
    dst_ref[...] = src_ref[...]


def kernel(inputs, ent_embeds):
    del inputs
    rows, dim = ent_embeds.shape
    wide = ent_embeds.T
    grid = (rows + _BLOCK_COLS - 1) // _BLOCK_COLS
    out = pl.pallas_call(
        _copy_body,
        out_shape=jax.ShapeDtypeStruct(wide.shape, wide.dtype),
        grid=(grid,),
        in_specs=[pl.BlockSpec((dim, _BLOCK_COLS), lambda i: (0, i))],
        out_specs=pl.BlockSpec((dim, _BLOCK_COLS), lambda i: (0, i)),
    )(wide)
    return out.T
